# baseline (speedup 1.0000x reference)
import functools

import jax
import jax.numpy as jnp
from jax import lax
from jax.experimental import pallas as pl
from jax.experimental.pallas import tpu as pltpu
from jax.experimental.pallas import tpu_sc as plsc

_NUM_LAYERS = 3
_N = 10000
_EMB = 128
_NNZ = 320000
_NC = 2
_NS = 16
_NW = _NC * _NS
_CHUNK = 128
_CPT0 = 80
_CPT1 = 80
_NCHUNKS = _NS * (_CPT0 + _CPT1)
_WIN = 40
_NPHASE = -(-_CPT0 // _WIN)
_NROWS_IDX = _NS * _CPT0 + (_NS - 1) * _CPT1 + _NPHASE * _WIN
_NNZ_PAD = _NS * (_CPT0 + _CPT1) * _CHUNK
_RPS = 632

_mesh = plsc.VectorSubcoreMesh(core_axis_name="c", subcore_axis_name="s")

_BCAST_DNUMS = lax.GatherDimensionNumbers(
    offset_dims=(), collapsed_slice_dims=(0,), start_index_map=(0,))


def _lane_bcast(v16, i):
    idx = jnp.full((16, 1), i, dtype=jnp.int32)
    return lax.gather(v16, idx, _BCAST_DNUMS, (1,),
                      mode=lax.GatherScatterMode.PROMISE_IN_BOUNDS)


@functools.partial(
    pl.kernel,
    out_type=jax.ShapeDtypeStruct((_NC, _N, _EMB), jnp.float32),
    mesh=_mesh,
    scratch_types=[
        pltpu.VMEM((_WIN, _CHUNK), jnp.int32),
        pltpu.VMEM((_WIN, _CHUNK), jnp.int32),
        pltpu.VMEM((_WIN, _CHUNK), jnp.float32),
        pltpu.VMEM((_CHUNK, _EMB), jnp.float32),
        pltpu.VMEM((_CHUNK, _EMB), jnp.float32),
        pltpu.VMEM_SHARED((_N, _EMB), jnp.float32),
        pltpu.SemaphoreType.DMA,
        pltpu.SemaphoreType.DMA,
        pltpu.SemaphoreType.DMA,
        pltpu.SemaphoreType.DMA,
    ],
)
def _spmm(table_hbm, gidx_hbm, sidx_hbm, vals_hbm, out_hbm,
          gidx_v, sidx_v, vals_v, rows_a, rows_b, acc_sh, gs_a, gs_b,
          ss_a, ss_b):
    c = lax.axis_index("c")
    s = lax.axis_index("s")
    base = jnp.where(c == 0, s * _CPT0, _NS * _CPT0 + s * _CPT1)
    cpt = jnp.where(c == 0, _CPT0, _CPT1)

    zero16 = jnp.zeros((16,), jnp.float32)

    @pl.loop(0, _CHUNK)
    def _zero_rows(r):
        for k in range(0, _EMB, 16):
            rows_a[r, pl.ds(k, 16)] = zero16

    row0 = jnp.minimum(s * _RPS, _N - _RPS)
    for j in range(4):
        pltpu.sync_copy(rows_a, acc_sh.at[pl.ds(row0 + j * _CHUNK, _CHUNK)])
    pltpu.sync_copy(rows_a.at[pl.ds(0, _RPS - 4 * _CHUNK)],
                    acc_sh.at[pl.ds(row0 + 4 * _CHUNK, _RPS - 4 * _CHUNK)])

    plsc.subcore_barrier()

    def _gather_start(g, rows, sem):
        pltpu.async_copy(table_hbm.at[gidx_v.at[g]], rows, sem)

    def _gather_wait(g, rows, sem):
        pltpu.make_async_copy(table_hbm.at[gidx_v.at[g]], rows, sem).wait()

    def _scale(g, rows):
        @pl.loop(0, _CHUNK, step=16)
        def _scale_16(r0):
            v16 = vals_v[g, pl.ds(r0, 16)]
            for i in range(16):
                vb = _lane_bcast(v16, i)
                for k in range(0, _EMB, 16):
                    rows[r0 + i, pl.ds(k, 16)] = rows[r0 + i, pl.ds(k, 16)] * vb

    def _scatter_start(g, rows, sem):
        return pltpu.async_copy(rows, acc_sh.at[sidx_v.at[g]], sem, add=True)

    for p in range(_NPHASE):
        pltpu.sync_copy(gidx_hbm.at[pl.ds(base + p * _WIN, _WIN)], gidx_v)
        pltpu.sync_copy(sidx_hbm.at[pl.ds(base + p * _WIN, _WIN)], sidx_v)
        pltpu.sync_copy(vals_hbm.at[pl.ds(base + p * _WIN, _WIN)], vals_v)
        nchunks = jnp.clip(cpt - p * _WIN, 0, _WIN)

        @pl.when(nchunks > 0)
        def _():
            _gather_start(0, rows_a, gs_a)
            _gather_start(1, rows_b, gs_b)

        @pl.loop(0, nchunks, step=2)
        def _chunk(g):
            _gather_wait(g, rows_a, gs_a)
            _scale(g, rows_a)
            sc_a = _scatter_start(g, rows_a, ss_a)

            _gather_wait(g + 1, rows_b, gs_b)
            _scale(g + 1, rows_b)

            sc_a.wait()

            @pl.when(g + 2 < nchunks)
            def _():
                _gather_start(g + 2, rows_a, gs_a)
            sc_b = _scatter_start(g + 1, rows_b, ss_b)
            sc_b.wait()

            @pl.when(g + 2 < nchunks)
            def _():
                _gather_start(g + 3, rows_b, gs_b)

    plsc.subcore_barrier()

    pltpu.sync_copy(acc_sh.at[pl.ds(row0, _RPS)],
                    out_hbm.at[c, pl.ds(row0, _RPS)])


_GRID = 5
_BLK = _N // _GRID


def _tc_add2(parts):
    def body(p_ref, o_ref):
        o_ref[...] = p_ref[0] + p_ref[1]

    return pl.pallas_call(
        body,
        grid=(_GRID,),
        in_specs=[pl.BlockSpec((2, _BLK, _EMB), lambda i: (0, i, 0))],
        out_specs=pl.BlockSpec((_BLK, _EMB), lambda i: (i, 0)),
        out_shape=jax.ShapeDtypeStruct((_N, _EMB), jnp.float32),
    )(parts)


def _tc_layer_first(parts, e_prev, att_b):
    def body(p_ref, e_ref, att_ref, oe_ref, oa_ref):
        prop = jnp.maximum(p_ref[0] + p_ref[1], 0.0)
        e_new = prop + e_ref[...]
        oe_ref[...] = e_new
        oa_ref[...] = att_ref[0] * e_ref[...] + att_ref[1] * e_new

    return pl.pallas_call(
        body,
        grid=(_GRID,),
        in_specs=[
            pl.BlockSpec((2, _BLK, _EMB), lambda i: (0, i, 0)),
            pl.BlockSpec((_BLK, _EMB), lambda i: (i, 0)),
            pl.BlockSpec((_NUM_LAYERS + 1, _EMB), lambda i: (0, 0)),
        ],
        out_specs=[
            pl.BlockSpec((_BLK, _EMB), lambda i: (i, 0)),
            pl.BlockSpec((_BLK, _EMB), lambda i: (i, 0)),
        ],
        out_shape=[
            jax.ShapeDtypeStruct((_N, _EMB), jnp.float32),
            jax.ShapeDtypeStruct((_N, _EMB), jnp.float32),
        ],
    )(parts, e_prev, att_b)


def _tc_layer_rest(parts, e_prev, acc_prev, att_b, layer):
    def body(p_ref, e_ref, a_ref, att_ref, oe_ref, oa_ref):
        prop = jnp.maximum(p_ref[0] + p_ref[1], 0.0)
        e_new = prop + e_ref[...]
        oe_ref[...] = e_new
        oa_ref[...] = a_ref[...] + att_ref[layer] * e_new

    return pl.pallas_call(
        body,
        grid=(_GRID,),
        in_specs=[
            pl.BlockSpec((2, _BLK, _EMB), lambda i: (0, i, 0)),
            pl.BlockSpec((_BLK, _EMB), lambda i: (i, 0)),
            pl.BlockSpec((_BLK, _EMB), lambda i: (i, 0)),
            pl.BlockSpec((_NUM_LAYERS + 1, _EMB), lambda i: (0, 0)),
        ],
        out_specs=[
            pl.BlockSpec((_BLK, _EMB), lambda i: (i, 0)),
            pl.BlockSpec((_BLK, _EMB), lambda i: (i, 0)),
        ],
        out_shape=[
            jax.ShapeDtypeStruct((_N, _EMB), jnp.float32),
            jax.ShapeDtypeStruct((_N, _EMB), jnp.float32),
        ],
    )(parts, e_prev, acc_prev, att_b)


def kernel(pois_embs, pad_all_train_sessions, hg_indices, hg_up_values,
           hg_pu_values, layer_attention):
    del pad_all_train_sessions
    idx = hg_indices.astype(jnp.int32)
    u_idx, p_idx = idx[0], idx[1]
    pad = _NROWS_IDX * _CHUNK - _NNZ
    spread = jnp.arange(pad, dtype=jnp.int32) % _N

    def prep_idx(a):
        return jnp.concatenate([a, spread]).reshape(_NROWS_IDX, _CHUNK)

    def prep_val(a):
        return jnp.concatenate(
            [a, jnp.zeros((pad,), a.dtype)]).reshape(_NROWS_IDX, _CHUNK)

    u2 = prep_idx(u_idx)
    p2 = prep_idx(p_idx)
    vup = prep_val(hg_up_values)
    vpu = prep_val(hg_pu_values)

    att = jax.nn.softmax(layer_attention.astype(jnp.float32), axis=0)
    att_b = jnp.broadcast_to(att[:, None], (_NUM_LAYERS + 1, _EMB))

    embs = pois_embs
    acc = None
    for layer in range(1, _NUM_LAYERS + 1):
        up_parts = _spmm(embs, p2, u2, vup)
        msg = _tc_add2(up_parts)
        dn_parts = _spmm(msg, u2, p2, vpu)
        if acc is None:
            embs, acc = _tc_layer_first(dn_parts, embs, att_b)
        else:
            embs, acc = _tc_layer_rest(dn_parts, embs, acc, att_b, layer)
    return acc


# speedup vs baseline: 1.0365x; 1.0365x over previous
import functools

import jax
import jax.numpy as jnp
from jax import lax
from jax.experimental import pallas as pl
from jax.experimental.pallas import tpu as pltpu
from jax.experimental.pallas import tpu_sc as plsc

_NUM_LAYERS = 3
_N = 10000
_EMB = 128
_NNZ = 320000
_NC = 2
_NS = 16
_NW = _NC * _NS
_CHUNK = 128
_CPT0 = 80
_CPT1 = 80
_NCHUNKS = _NS * (_CPT0 + _CPT1)
_WIN = 40
_NPHASE = -(-_CPT0 // _WIN)
_NROWS_IDX = _NS * _CPT0 + (_NS - 1) * _CPT1 + _NPHASE * _WIN
_NNZ_PAD = _NS * (_CPT0 + _CPT1) * _CHUNK
_RPS = 632

_mesh = plsc.VectorSubcoreMesh(core_axis_name="c", subcore_axis_name="s")

_BCAST_DNUMS = lax.GatherDimensionNumbers(
    offset_dims=(), collapsed_slice_dims=(0,), start_index_map=(0,))


def _lane_bcast(v16, i):
    idx = jnp.full((16, 1), i, dtype=jnp.int32)
    return lax.gather(v16, idx, _BCAST_DNUMS, (1,),
                      mode=lax.GatherScatterMode.PROMISE_IN_BOUNDS)


@functools.partial(
    pl.kernel,
    out_type=jax.ShapeDtypeStruct((_NC, _N, _EMB), jnp.float32),
    mesh=_mesh,
    scratch_types=[
        pltpu.VMEM((_WIN, _CHUNK), jnp.int32),
        pltpu.VMEM((_WIN, _CHUNK), jnp.int32),
        pltpu.VMEM((_WIN, _CHUNK), jnp.float32),
        pltpu.VMEM((_CHUNK, _EMB), jnp.float32),
        pltpu.VMEM((_CHUNK, _EMB), jnp.float32),
        pltpu.VMEM_SHARED((_N, _EMB), jnp.float32),
        pltpu.SemaphoreType.DMA,
        pltpu.SemaphoreType.DMA,
    ],
)
def _spmm(table_hbm, gidx_hbm, sidx_hbm, vals_hbm, out_hbm,
          gidx_v, sidx_v, vals_v, rows_a, rows_b, acc_sh, gs_a, gs_b):
    c = lax.axis_index("c")
    s = lax.axis_index("s")
    base = jnp.where(c == 0, s * _CPT0, _NS * _CPT0 + s * _CPT1)
    cpt = jnp.where(c == 0, _CPT0, _CPT1)

    zero16 = jnp.zeros((16,), jnp.float32)

    @pl.loop(0, _CHUNK)
    def _zero_rows(r):
        for k in range(0, _EMB, 16):
            rows_a[r, pl.ds(k, 16)] = zero16

    row0 = jnp.minimum(s * _RPS, _N - _RPS)
    for j in range(4):
        pltpu.sync_copy(rows_a, acc_sh.at[pl.ds(row0 + j * _CHUNK, _CHUNK)])
    pltpu.sync_copy(rows_a.at[pl.ds(0, _RPS - 4 * _CHUNK)],
                    acc_sh.at[pl.ds(row0 + 4 * _CHUNK, _RPS - 4 * _CHUNK)])

    plsc.subcore_barrier()

    def _gather_start(g, rows, sem):
        pltpu.async_copy(table_hbm.at[gidx_v.at[g]], rows, sem)

    def _gather_wait(g, rows, sem):
        pltpu.make_async_copy(table_hbm.at[gidx_v.at[g]], rows, sem).wait()

    def _scale(g, rows):
        @pl.loop(0, _CHUNK, step=16)
        def _scale_16(r0):
            v16 = vals_v[g, pl.ds(r0, 16)]
            for i in range(16):
                vb = _lane_bcast(v16, i)
                for k in range(0, _EMB, 16):
                    rows[r0 + i, pl.ds(k, 16)] = rows[r0 + i, pl.ds(k, 16)] * vb

    for p in range(_NPHASE):
        pltpu.sync_copy(gidx_hbm.at[pl.ds(base + p * _WIN, _WIN)], gidx_v)
        pltpu.sync_copy(sidx_hbm.at[pl.ds(base + p * _WIN, _WIN)], sidx_v)
        pltpu.sync_copy(vals_hbm.at[pl.ds(base + p * _WIN, _WIN)], vals_v)
        nchunks = jnp.clip(cpt - p * _WIN, 0, _WIN)

        @pl.when(nchunks > 0)
        def _():
            _gather_start(0, rows_a, gs_a)

        @pl.loop(0, nchunks, step=2)
        def _chunk(g):
            _gather_wait(g, rows_a, gs_a)
            _gather_start(g + 1, rows_b, gs_b)
            _scale(g, rows_a)
            pltpu.sync_copy(rows_a, acc_sh.at[sidx_v.at[g]], add=True)

            _gather_wait(g + 1, rows_b, gs_b)

            @pl.when(g + 2 < nchunks)
            def _():
                _gather_start(g + 2, rows_a, gs_a)
            _scale(g + 1, rows_b)
            pltpu.sync_copy(rows_b, acc_sh.at[sidx_v.at[g + 1]], add=True)

    plsc.subcore_barrier()

    pltpu.sync_copy(acc_sh.at[pl.ds(row0, _RPS)],
                    out_hbm.at[c, pl.ds(row0, _RPS)])


_GRID = 5
_BLK = _N // _GRID


def _tc_add2(parts):
    def body(p_ref, o_ref):
        o_ref[...] = p_ref[0] + p_ref[1]

    return pl.pallas_call(
        body,
        grid=(_GRID,),
        in_specs=[pl.BlockSpec((2, _BLK, _EMB), lambda i: (0, i, 0))],
        out_specs=pl.BlockSpec((_BLK, _EMB), lambda i: (i, 0)),
        out_shape=jax.ShapeDtypeStruct((_N, _EMB), jnp.float32),
    )(parts)


def _tc_layer_first(parts, e_prev, att_b):
    def body(p_ref, e_ref, att_ref, oe_ref, oa_ref):
        prop = jnp.maximum(p_ref[0] + p_ref[1], 0.0)
        e_new = prop + e_ref[...]
        oe_ref[...] = e_new
        oa_ref[...] = att_ref[0] * e_ref[...] + att_ref[1] * e_new

    return pl.pallas_call(
        body,
        grid=(_GRID,),
        in_specs=[
            pl.BlockSpec((2, _BLK, _EMB), lambda i: (0, i, 0)),
            pl.BlockSpec((_BLK, _EMB), lambda i: (i, 0)),
            pl.BlockSpec((_NUM_LAYERS + 1, _EMB), lambda i: (0, 0)),
        ],
        out_specs=[
            pl.BlockSpec((_BLK, _EMB), lambda i: (i, 0)),
            pl.BlockSpec((_BLK, _EMB), lambda i: (i, 0)),
        ],
        out_shape=[
            jax.ShapeDtypeStruct((_N, _EMB), jnp.float32),
            jax.ShapeDtypeStruct((_N, _EMB), jnp.float32),
        ],
    )(parts, e_prev, att_b)


def _tc_layer_rest(parts, e_prev, acc_prev, att_b, layer):
    def body(p_ref, e_ref, a_ref, att_ref, oe_ref, oa_ref):
        prop = jnp.maximum(p_ref[0] + p_ref[1], 0.0)
        e_new = prop + e_ref[...]
        oe_ref[...] = e_new
        oa_ref[...] = a_ref[...] + att_ref[layer] * e_new

    return pl.pallas_call(
        body,
        grid=(_GRID,),
        in_specs=[
            pl.BlockSpec((2, _BLK, _EMB), lambda i: (0, i, 0)),
            pl.BlockSpec((_BLK, _EMB), lambda i: (i, 0)),
            pl.BlockSpec((_BLK, _EMB), lambda i: (i, 0)),
            pl.BlockSpec((_NUM_LAYERS + 1, _EMB), lambda i: (0, 0)),
        ],
        out_specs=[
            pl.BlockSpec((_BLK, _EMB), lambda i: (i, 0)),
            pl.BlockSpec((_BLK, _EMB), lambda i: (i, 0)),
        ],
        out_shape=[
            jax.ShapeDtypeStruct((_N, _EMB), jnp.float32),
            jax.ShapeDtypeStruct((_N, _EMB), jnp.float32),
        ],
    )(parts, e_prev, acc_prev, att_b)


def kernel(pois_embs, pad_all_train_sessions, hg_indices, hg_up_values,
           hg_pu_values, layer_attention):
    del pad_all_train_sessions
    idx = hg_indices.astype(jnp.int32)
    u_idx, p_idx = idx[0], idx[1]
    pad = _NROWS_IDX * _CHUNK - _NNZ
    spread = jnp.arange(pad, dtype=jnp.int32) % _N

    def prep_idx(a):
        return jnp.concatenate([a, spread]).reshape(_NROWS_IDX, _CHUNK)

    def prep_val(a):
        return jnp.concatenate(
            [a, jnp.zeros((pad,), a.dtype)]).reshape(_NROWS_IDX, _CHUNK)

    u2 = prep_idx(u_idx)
    p2 = prep_idx(p_idx)
    vup = prep_val(hg_up_values)
    vpu = prep_val(hg_pu_values)

    att = jax.nn.softmax(layer_attention.astype(jnp.float32), axis=0)
    att_b = jnp.broadcast_to(att[:, None], (_NUM_LAYERS + 1, _EMB))

    embs = pois_embs
    acc = None
    for layer in range(1, _NUM_LAYERS + 1):
        up_parts = _spmm(embs, p2, u2, vup)
        msg = _tc_add2(up_parts)
        dn_parts = _spmm(msg, u2, p2, vpu)
        if acc is None:
            embs, acc = _tc_layer_first(dn_parts, embs, att_b)
        else:
            embs, acc = _tc_layer_rest(dn_parts, embs, acc, att_b, layer)
    return acc
